# baseline (speedup 1.0000x reference)
import jax
import jax.numpy as jnp
from jax import lax
from jax.experimental import pallas as pl
from jax.experimental.pallas import tpu as pltpu
from jax.experimental.pallas import tpu_sc as plsc

N = 10000
NPAD = 10240
E = 320000
F = 128
DE = 16
G = 64
NC = 2
NS = 16
L = 16
NW = NC * NS
B = 40
NCH = 250
EP = NW * B * NCH
EWP = B * NCH
TPAD = 48
RPT = NPAD // NS
ZR = 32
RBLK = 2048
MBLK = 2000


def _mm_bias_body(x_ref, w_ref, b_ref, o_ref):
    o_ref[...] = jnp.dot(x_ref[...], w_ref[...],
                         preferred_element_type=jnp.float32) + b_ref[...]


def _mm_bias(x, w, b):
    n, k = x.shape
    m = w.shape[1]
    return pl.pallas_call(
        _mm_bias_body,
        grid=(n // MBLK,),
        in_specs=[pl.BlockSpec((MBLK, k), lambda i: (i, 0)),
                  pl.BlockSpec((k, m), lambda i: (0, 0)),
                  pl.BlockSpec((1, m), lambda i: (0, 0))],
        out_specs=pl.BlockSpec((MBLK, m), lambda i: (i, 0)),
        out_shape=jax.ShapeDtypeStruct((n, m), jnp.float32),
    )(x, w, b.reshape(1, m))


_EBLK = 4000 if EP % 4000 == 0 else 4032


_PERM = [32 * g + (m // 2 if m % 2 == 0 else 16 + (m - 1) // 2)
         for g in range(4) for m in range(32)]


def _ef_body(ea_ref, w1_ref, o1_ref):
    o1_ref[...] = jnp.dot(ea_ref[...], w1_ref[...],
                          preferred_element_type=jnp.float32
                          ).astype(jnp.bfloat16)


def _ef_one(edge_attr, we):
    return pl.pallas_call(
        _ef_body,
        grid=(EP // _EBLK,),
        in_specs=[pl.BlockSpec((_EBLK, DE), lambda i: (i, 0)),
                  pl.BlockSpec((DE, F), lambda i: (0, 0))],
        out_specs=pl.BlockSpec((_EBLK, F), lambda i: (i, 0)),
        out_shape=jax.ShapeDtypeStruct((EP, F), jnp.bfloat16),
    )(edge_attr, we)


def _prep_body(ea_ref, we1_ref, x_ref, wl_ref, bl_ref, wr_ref, br_ref,
               oe_ref, ol_ref, or_ref):
    i = pl.program_id(0)
    oe_ref[...] = jnp.dot(ea_ref[...], we1_ref[...],
                          preferred_element_type=jnp.float32
                          ).astype(jnp.bfloat16)

    @pl.when(i < N // MBLK)
    def _():
        xb = x_ref[...]
        ol_ref[...] = jnp.dot(xb, wl_ref[...],
                              preferred_element_type=jnp.float32) + bl_ref[...]
        or_ref[...] = jnp.dot(xb, wr_ref[...],
                              preferred_element_type=jnp.float32) + br_ref[...]


def _prep(edge_attr, we1, x, wl, bl, wr, br):
    nblk = N // MBLK
    cap = lambda i: jnp.minimum(i, nblk - 1)
    return pl.pallas_call(
        _prep_body,
        grid=(EP // _EBLK,),
        in_specs=[pl.BlockSpec((_EBLK, DE), lambda i: (i, 0)),
                  pl.BlockSpec((DE, F), lambda i: (0, 0)),
                  pl.BlockSpec((MBLK, F), lambda i: (cap(i), 0)),
                  pl.BlockSpec((F, F), lambda i: (0, 0)),
                  pl.BlockSpec((1, F), lambda i: (0, 0)),
                  pl.BlockSpec((F, F), lambda i: (0, 0)),
                  pl.BlockSpec((1, F), lambda i: (0, 0))],
        out_specs=[pl.BlockSpec((_EBLK, F), lambda i: (i, 0)),
                   pl.BlockSpec((MBLK, F), lambda i: (cap(i), 0)),
                   pl.BlockSpec((MBLK, F), lambda i: (cap(i), 0))],
        out_shape=[jax.ShapeDtypeStruct((EP, F), jnp.bfloat16),
                   jax.ShapeDtypeStruct((N, F), jnp.float32),
                   jax.ShapeDtypeStruct((N, F), jnp.float32)],
    )(edge_attr, we1, x, wl, bl.reshape(1, F), wr, br.reshape(1, F))


def _fuse_h(acc_ref, den_ref, b_ref):
    num = acc_ref[0] + acc_ref[1]
    den16 = den_ref[0] + den_ref[1]
    rows = lax.broadcasted_iota(jnp.int32, (RBLK, RBLK // F), 0) // F
    p = (rows == lax.broadcasted_iota(jnp.int32, (RBLK, RBLK // F), 1))
    d1 = jnp.dot(p.astype(jnp.float32), den16,
                 preferred_element_type=jnp.float32)
    m = (lax.broadcasted_iota(jnp.int32, (RBLK, F), 0) % F
         == lax.broadcasted_iota(jnp.int32, (RBLK, F), 1))
    den = jnp.sum(jnp.where(m, d1, 0.0), axis=1, keepdims=True)
    return jnp.maximum(num / (den + 1e-16) + b_ref[...], 0.0)


def _mid_body(acc_ref, den_ref, b1_ref, wl_ref, bl_ref, wr_ref, br_ref,
              ol_ref, or_ref):
    h = _fuse_h(acc_ref, den_ref, b1_ref)
    ol_ref[...] = jnp.dot(h, wl_ref[...],
                          preferred_element_type=jnp.float32) + bl_ref[...]
    or_ref[...] = jnp.dot(h, wr_ref[...],
                          preferred_element_type=jnp.float32) + br_ref[...]


def _mid(acc, den3, b1, wl, bl, wr, br):
    return pl.pallas_call(
        _mid_body,
        grid=(NPAD // RBLK,),
        in_specs=[pl.BlockSpec((NC, RBLK, F), lambda i: (0, i, 0)),
                  pl.BlockSpec((NC, RBLK // F, F), lambda i: (0, i, 0)),
                  pl.BlockSpec((1, F), lambda i: (0, 0)),
                  pl.BlockSpec((F, F), lambda i: (0, 0)),
                  pl.BlockSpec((1, F), lambda i: (0, 0)),
                  pl.BlockSpec((F, F), lambda i: (0, 0)),
                  pl.BlockSpec((1, F), lambda i: (0, 0))],
        out_specs=[pl.BlockSpec((RBLK, F), lambda i: (i, 0)),
                   pl.BlockSpec((RBLK, F), lambda i: (i, 0))],
        out_shape=[jax.ShapeDtypeStruct((NPAD, F), jnp.float32),
                   jax.ShapeDtypeStruct((NPAD, F), jnp.float32)],
    )(acc, den3, b1.reshape(1, F), wl, bl.reshape(1, F), wr, br.reshape(1, F))


def _final_body(acc_ref, den_ref, b2_ref, batch_ref, wfc_ref, bfc_ref, o_ref,
                psum_ref, pcnt_ref):
    i = pl.program_id(0)
    h = _fuse_h(acc_ref, den_ref, b2_ref)
    seg = batch_ref[...].reshape(RBLK, 1)
    onehot = (seg == lax.iota(jnp.int32, G).reshape(1, G)).astype(jnp.float32)
    ps = lax.dot_general(onehot, h, (((0,), (0,)), ((), ())),
                         preferred_element_type=jnp.float32)
    pc = lax.dot_general(onehot, jnp.ones((RBLK, F), jnp.float32),
                         (((0,), (0,)), ((), ())),
                         preferred_element_type=jnp.float32)

    @pl.when(i == 0)
    def _():
        psum_ref[...] = ps
        pcnt_ref[...] = pc

    @pl.when(i > 0)
    def _():
        psum_ref[...] += ps
        pcnt_ref[...] += pc

    @pl.when(i == pl.num_programs(0) - 1)
    def _():
        pooled = psum_ref[...] / jnp.maximum(pcnt_ref[...], 1.0)
        o_ref[...] = jnp.dot(pooled, wfc_ref[...],
                             preferred_element_type=jnp.float32) + bfc_ref[...]


def _final(acc, den3, b2, batch3, wfc, bfc):
    return pl.pallas_call(
        _final_body,
        grid=(NPAD // RBLK,),
        in_specs=[pl.BlockSpec((NC, RBLK, F), lambda i: (0, i, 0)),
                  pl.BlockSpec((NC, RBLK // F, F), lambda i: (0, i, 0)),
                  pl.BlockSpec((1, F), lambda i: (0, 0)),
                  pl.BlockSpec((1, 1, RBLK), lambda i: (i, 0, 0)),
                  pl.BlockSpec((F, F), lambda i: (0, 0)),
                  pl.BlockSpec((1, F), lambda i: (0, 0))],
        out_specs=pl.BlockSpec((G, F), lambda i: (0, 0)),
        out_shape=jax.ShapeDtypeStruct((G, F), jnp.float32),
        scratch_shapes=[pltpu.VMEM((G, F), jnp.float32),
                        pltpu.VMEM((G, F), jnp.float32)],
    )(acc, den3, b2.reshape(1, F), batch3, wfc, bfc.reshape(1, F))


def _edge_body(xl_hbm, xr_hbm, ef_hbm, src_hbm, dst_hbm, att_hbm,
               acc_hbm, den_hbm,
               src0_v, dst0_v, xj0_v, xi0_v, ef0_v,
               src1_v, dst1_v, xj1_v, xi1_v, ef1_v,
               dsc0_v, dsc1_v, w0_v, w1_v,
               t_v, att_v, zb_v, zd_v,
               acc_s, den_s, si0, si1, sg0, sg1, ss0, ss1):
    cid = lax.axis_index("c")
    sid = lax.axis_index("s")
    wid = cid * NS + sid
    sets = ((src0_v, dst0_v, xj0_v, xi0_v, ef0_v, dsc0_v, w0_v, si0, sg0, ss0),
            (src1_v, dst1_v, xj1_v, xi1_v, ef1_v, dsc1_v, w1_v, si1, sg1, ss1))

    def zr_body(r, carry):
        for k2 in range(F // L):
            zb_v[r, pl.ds(k2 * L, L)] = jnp.zeros((L,), jnp.float32)
        return carry

    lax.fori_loop(0, ZR, zr_body, 0)

    def zd_body(r, carry):
        zd_v[pl.ds(r * L, L)] = jnp.zeros((L,), jnp.float32)
        return carry

    lax.fori_loop(0, RPT // L, zd_body, 0)
    pltpu.sync_copy(zd_v, den_s.at[pl.ds(sid * RPT, RPT)])

    def zc_body(i, carry):
        pltpu.sync_copy(zb_v, acc_s.at[pl.ds(sid * RPT + i * ZR, ZR)])
        return carry

    lax.fori_loop(0, RPT // ZR, zc_body, 0)
    pltpu.sync_copy(att_hbm, att_v)
    for r in range(B, TPAD):
        t_v[r, :] = jnp.zeros((L,), jnp.float32)
    plsc.subcore_barrier()

    ebase = wid * EWP
    fiota = lax.iota(jnp.int32, L)
    att8 = tuple(att_v[pl.ds(k * L, L)] for k in range(F // L))

    def _issue_idx(c, s):
        src_v, dst_v = s[0], s[1]
        si = s[7]
        e0 = ebase + c * B
        pltpu.async_copy(src_hbm.at[pl.ds(e0, B)], src_v, si)
        pltpu.async_copy(dst_hbm.at[pl.ds(e0, B)], dst_v, si)

    def _wait_idx(c, s):
        src_v, dst_v = s[0], s[1]
        si = s[7]
        e0 = ebase + c * B
        pltpu.make_async_copy(src_hbm.at[pl.ds(e0, B)], src_v, si).wait()
        pltpu.make_async_copy(dst_hbm.at[pl.ds(e0, B)], dst_v, si).wait()

    def _issue_gather(c, s):
        src_v, dst_v, xj_v, xi_v, ef_v = s[0], s[1], s[2], s[3], s[4]
        sg = s[8]
        e0 = ebase + c * B
        pltpu.async_copy(xl_hbm.at[src_v], xj_v, sg)
        pltpu.async_copy(xr_hbm.at[dst_v], xi_v, sg)
        pltpu.async_copy(ef_hbm.at[pl.ds(e0, B)], ef_v, sg)

    def _wait_gather(c, s):
        src_v, dst_v, xj_v, xi_v, ef_v = s[0], s[1], s[2], s[3], s[4]
        sg = s[8]
        e0 = ebase + c * B
        pltpu.make_async_copy(xl_hbm.at[src_v], xj_v, sg).wait()
        pltpu.make_async_copy(xr_hbm.at[dst_v], xi_v, sg).wait()
        pltpu.make_async_copy(ef_hbm.at[pl.ds(e0, B)], ef_v, sg).wait()

    def _wait_scatter(s):
        xj_v, dsc_v, w_v, ss = s[2], s[5], s[6], s[9]
        pltpu.make_async_copy(xj_v, acc_s.at[dsc_v], ss).wait()
        pltpu.make_async_copy(w_v.at[pl.ds(0, B)], den_s.at[dsc_v], ss).wait()

    def _process(c, this, nxt, att8c):
        src_v, dst_v, xj_v, xi_v, ef_v, dsc_v, w_v = this[:7]
        ss = this[9]
        _wait_gather(c, this)

        @pl.when(c + 1 < NCH)
        def _():
            @pl.when(c >= 1)
            def _():
                _wait_scatter(nxt)
            _wait_idx(c + 1, nxt)
            _issue_gather(c + 1, nxt)

        @plsc.parallel_loop(0, B, step=1, unroll=4, carry=att8c)
        def e_loop(e, att8i):
            t = jnp.zeros((L,), jnp.float32)
            for k2 in range(F // (2 * L)):
                eu = plsc.bitcast(ef_v[e, pl.ds(k2 * 2 * L, 2 * L)],
                                  jnp.uint32)
                efa = plsc.bitcast(eu << 16, jnp.float32)
                efb = plsc.bitcast(eu & jnp.uint32(0xFFFF0000), jnp.float32)
                for (k, ef) in ((2 * k2, efa), (2 * k2 + 1, efb)):
                    s = (xi_v[e, pl.ds(k * L, L)] + xj_v[e, pl.ds(k * L, L)]
                         + ef)
                    m = jnp.where(s >= 0.0, s, 0.2 * s)
                    t = t + m * att8i[k]
            t_v[e, :] = t
            return att8i

        att8c = e_loop

        for g in range(TPAD // L):
            rows = g * L + fiota
            lsum = jnp.zeros((L,), jnp.float32)
            for c16 in range(L):
                cols = jnp.full((L,), c16, jnp.int32)
                lsum = lsum + plsc.load_gather(t_v, [rows, cols])
            w_v[pl.ds(g * L, L)] = jnp.exp(lsum)

        @plsc.parallel_loop(0, B, step=1, unroll=4)
        def s_loop(e):
            w = w_v[pl.ds(e, L)][0]
            for k in range(F // L):
                xj_v[e, pl.ds(k * L, L)] = xj_v[e, pl.ds(k * L, L)] * w
        dsc_v[pl.ds(0, L)] = dst_v[pl.ds(0, L)]
        dsc_v[pl.ds(L, L)] = dst_v[pl.ds(L, L)]
        dsc_v[pl.ds(B - L, L)] = dst_v[pl.ds(B - L, L)]
        ss = this[9]
        pltpu.async_copy(xj_v, acc_s.at[dsc_v], ss, add=True)
        pltpu.async_copy(w_v.at[pl.ds(0, B)], den_s.at[dsc_v], ss, add=True)

        @pl.when(c + 2 < NCH)
        def _():
            _issue_idx(c + 2, this)

        return att8c

    _issue_idx(0, sets[0])
    _wait_idx(0, sets[0])
    _issue_gather(0, sets[0])
    _issue_idx(1, sets[1])

    def pair_body(cc, att8c):
        att8c = _process(2 * cc, sets[0], sets[1], att8c)
        att8c = _process(2 * cc + 1, sets[1], sets[0], att8c)
        return att8c

    lax.fori_loop(0, NCH // 2, pair_body, att8)
    _wait_scatter(sets[0])
    _wait_scatter(sets[1])
    plsc.subcore_barrier()

    def pub_body(i, carry):
        r0 = sid * RPT + i * ZR
        pltpu.sync_copy(acc_s.at[pl.ds(r0, ZR)], zb_v)
        pltpu.sync_copy(zb_v, acc_hbm.at[cid, pl.ds(r0, ZR)])
        return carry

    lax.fori_loop(0, RPT // ZR, pub_body, 0)
    pltpu.sync_copy(den_s.at[pl.ds(sid * RPT, RPT)], zd_v)
    pltpu.sync_copy(zd_v, den_hbm.at[cid, pl.ds(sid * RPT, RPT)])


def _edge_pass(xl, xr, ef, src, dst, att):
    mesh = plsc.VectorSubcoreMesh(core_axis_name="c", subcore_axis_name="s",
                                  num_cores=NC, num_subcores=NS)
    buf_set = [
        pltpu.VMEM((B,), jnp.int32),
        pltpu.VMEM((B,), jnp.int32),
        pltpu.VMEM((B, F), jnp.float32),
        pltpu.VMEM((B, F), jnp.float32),
        pltpu.VMEM((B, F), jnp.bfloat16),
    ]
    kern = pl.kernel(
        _edge_body,
        out_type=(jax.ShapeDtypeStruct((NC, NPAD, F), jnp.float32),
                  jax.ShapeDtypeStruct((NC, NPAD), jnp.float32)),
        mesh=mesh,
        scratch_types=buf_set + buf_set + [
            pltpu.VMEM((B,), jnp.int32),
            pltpu.VMEM((B,), jnp.int32),
            pltpu.VMEM((TPAD + L,), jnp.float32),
            pltpu.VMEM((TPAD + L,), jnp.float32),
            pltpu.VMEM((TPAD, L), jnp.float32),
            pltpu.VMEM((F,), jnp.float32),
            pltpu.VMEM((ZR, F), jnp.float32),
            pltpu.VMEM((RPT,), jnp.float32),
            pltpu.VMEM_SHARED((NPAD, F), jnp.float32),
            pltpu.VMEM_SHARED((NPAD,), jnp.float32),
            pltpu.SemaphoreType.DMA,
            pltpu.SemaphoreType.DMA,
            pltpu.SemaphoreType.DMA,
            pltpu.SemaphoreType.DMA,
            pltpu.SemaphoreType.DMA,
            pltpu.SemaphoreType.DMA,
        ],
        compiler_params=pltpu.CompilerParams(use_tc_tiling_on_sc=False,
                                             needs_layout_passes=False),
    )
    return kern(xl, xr, ef, src, dst, att)


def kernel(x, edge_index, edge_attr, batch,
           Wl1, bl1, Wr1, br1, We1, att1, bias1,
           Wl2, bl2, Wr2, br2, We2, att2, bias2,
           Wfc, bfc):
    src = edge_index[0].astype(jnp.int32)
    dst = edge_index[1].astype(jnp.int32)
    ea_pad = edge_attr
    if EP > E:
        src = jnp.concatenate([src, jnp.zeros((EP - E,), jnp.int32)])
        dst = jnp.concatenate([dst, jnp.full((EP - E,), NPAD - 1, jnp.int32)])
        ea_pad = jnp.concatenate([edge_attr,
                                  jnp.zeros((EP - E, DE), jnp.float32)])
    batch_pad = jnp.concatenate(
        [batch.astype(jnp.int32), jnp.full((NPAD - N,), G, jnp.int32)])
    batch3 = batch_pad.reshape(NPAD // RBLK, 1, RBLK)

    perm = jnp.asarray(_PERM, dtype=jnp.int32)
    ef1, xl1, xr1 = _prep(ea_pad, We1[:, perm], x, Wl1, bl1, Wr1, br1)
    if EP > E:
        zrows = jnp.zeros((NPAD - N, F), jnp.float32)
        xl1 = jnp.concatenate([xl1, zrows])
        xr1 = jnp.concatenate([xr1, zrows])
    ef2 = _ef_one(ea_pad, We2[:, perm])

    acc1, den1 = _edge_pass(xl1, xr1, ef1, src, dst, att1.reshape(F))
    xl2, xr2 = _mid(acc1, den1.reshape(NC, NPAD // F, F),
                    bias1, Wl2, bl2, Wr2, br2)
    acc2, den2 = _edge_pass(xl2, xr2, ef2, src, dst, att2.reshape(F))
    return _final(acc2, den2.reshape(NC, NPAD // F, F),
                  bias2, batch3, Wfc, bfc)


# speedup vs baseline: 1.2795x; 1.2795x over previous
import jax
import jax.numpy as jnp
from jax import lax
from jax.experimental import pallas as pl
from jax.experimental.pallas import tpu as pltpu
from jax.experimental.pallas import tpu_sc as plsc

N = 10000
NPAD = 10240
E = 320000
F = 128
DE = 16
G = 64
NC = 2
NS = 16
L = 16
NW = NC * NS
B = 40
NCH = 250
EP = NW * B * NCH
EWP = B * NCH
TPAD = 48
RPT = NPAD // NS
ZR = 32
RBLK = 2048
MBLK = 2000


def _mm_bias_body(x_ref, w_ref, b_ref, o_ref):
    o_ref[...] = jnp.dot(x_ref[...], w_ref[...],
                         preferred_element_type=jnp.float32) + b_ref[...]


def _mm_bias(x, w, b):
    n, k = x.shape
    m = w.shape[1]
    return pl.pallas_call(
        _mm_bias_body,
        grid=(n // MBLK,),
        in_specs=[pl.BlockSpec((MBLK, k), lambda i: (i, 0)),
                  pl.BlockSpec((k, m), lambda i: (0, 0)),
                  pl.BlockSpec((1, m), lambda i: (0, 0))],
        out_specs=pl.BlockSpec((MBLK, m), lambda i: (i, 0)),
        out_shape=jax.ShapeDtypeStruct((n, m), jnp.float32),
    )(x, w, b.reshape(1, m))


_EBLK = 4000 if EP % 4000 == 0 else 4032


def _ef_body(ea_ref, w1_ref, o1_ref):
    o1_ref[...] = jnp.dot(ea_ref[...], w1_ref[...],
                          preferred_element_type=jnp.float32)


def _ef_one(edge_attr, we):
    return pl.pallas_call(
        _ef_body,
        grid=(EP // _EBLK,),
        in_specs=[pl.BlockSpec((_EBLK, DE), lambda i: (i, 0)),
                  pl.BlockSpec((DE, F), lambda i: (0, 0))],
        out_specs=pl.BlockSpec((_EBLK, F), lambda i: (i, 0)),
        out_shape=jax.ShapeDtypeStruct((EP, F), jnp.float32),
    )(edge_attr, we)


def _prep_body(ea_ref, we1_ref, x_ref, wl_ref, bl_ref, wr_ref, br_ref,
               oe_ref, ol_ref, or_ref):
    i = pl.program_id(0)
    oe_ref[...] = jnp.dot(ea_ref[...], we1_ref[...],
                          preferred_element_type=jnp.float32)

    @pl.when(i < N // MBLK)
    def _():
        xb = x_ref[...]
        ol_ref[...] = jnp.dot(xb, wl_ref[...],
                              preferred_element_type=jnp.float32) + bl_ref[...]
        or_ref[...] = jnp.dot(xb, wr_ref[...],
                              preferred_element_type=jnp.float32) + br_ref[...]


def _prep(edge_attr, we1, x, wl, bl, wr, br):
    nblk = N // MBLK
    cap = lambda i: jnp.minimum(i, nblk - 1)
    return pl.pallas_call(
        _prep_body,
        grid=(EP // _EBLK,),
        in_specs=[pl.BlockSpec((_EBLK, DE), lambda i: (i, 0)),
                  pl.BlockSpec((DE, F), lambda i: (0, 0)),
                  pl.BlockSpec((MBLK, F), lambda i: (cap(i), 0)),
                  pl.BlockSpec((F, F), lambda i: (0, 0)),
                  pl.BlockSpec((1, F), lambda i: (0, 0)),
                  pl.BlockSpec((F, F), lambda i: (0, 0)),
                  pl.BlockSpec((1, F), lambda i: (0, 0))],
        out_specs=[pl.BlockSpec((_EBLK, F), lambda i: (i, 0)),
                   pl.BlockSpec((MBLK, F), lambda i: (cap(i), 0)),
                   pl.BlockSpec((MBLK, F), lambda i: (cap(i), 0))],
        out_shape=[jax.ShapeDtypeStruct((EP, F), jnp.float32),
                   jax.ShapeDtypeStruct((N, F), jnp.float32),
                   jax.ShapeDtypeStruct((N, F), jnp.float32)],
    )(edge_attr, we1, x, wl, bl.reshape(1, F), wr, br.reshape(1, F))


def _fuse_h(acc_ref, den_ref, b_ref):
    num = acc_ref[0] + acc_ref[1]
    den16 = den_ref[0] + den_ref[1]
    rows = lax.broadcasted_iota(jnp.int32, (RBLK, RBLK // F), 0) // F
    p = (rows == lax.broadcasted_iota(jnp.int32, (RBLK, RBLK // F), 1))
    d1 = jnp.dot(p.astype(jnp.float32), den16,
                 preferred_element_type=jnp.float32)
    m = (lax.broadcasted_iota(jnp.int32, (RBLK, F), 0) % F
         == lax.broadcasted_iota(jnp.int32, (RBLK, F), 1))
    den = jnp.sum(jnp.where(m, d1, 0.0), axis=1, keepdims=True)
    return jnp.maximum(num / (den + 1e-16) + b_ref[...], 0.0)


def _mid_body(acc_ref, den_ref, b1_ref, wl_ref, bl_ref, wr_ref, br_ref,
              ol_ref, or_ref):
    h = _fuse_h(acc_ref, den_ref, b1_ref)
    ol_ref[...] = jnp.dot(h, wl_ref[...],
                          preferred_element_type=jnp.float32) + bl_ref[...]
    or_ref[...] = jnp.dot(h, wr_ref[...],
                          preferred_element_type=jnp.float32) + br_ref[...]


def _mid(acc, den3, b1, wl, bl, wr, br):
    return pl.pallas_call(
        _mid_body,
        grid=(NPAD // RBLK,),
        in_specs=[pl.BlockSpec((NC, RBLK, F), lambda i: (0, i, 0)),
                  pl.BlockSpec((NC, RBLK // F, F), lambda i: (0, i, 0)),
                  pl.BlockSpec((1, F), lambda i: (0, 0)),
                  pl.BlockSpec((F, F), lambda i: (0, 0)),
                  pl.BlockSpec((1, F), lambda i: (0, 0)),
                  pl.BlockSpec((F, F), lambda i: (0, 0)),
                  pl.BlockSpec((1, F), lambda i: (0, 0))],
        out_specs=[pl.BlockSpec((RBLK, F), lambda i: (i, 0)),
                   pl.BlockSpec((RBLK, F), lambda i: (i, 0))],
        out_shape=[jax.ShapeDtypeStruct((NPAD, F), jnp.float32),
                   jax.ShapeDtypeStruct((NPAD, F), jnp.float32)],
    )(acc, den3, b1.reshape(1, F), wl, bl.reshape(1, F), wr, br.reshape(1, F))


def _final_body(acc_ref, den_ref, b2_ref, batch_ref, wfc_ref, bfc_ref, o_ref,
                psum_ref, pcnt_ref):
    i = pl.program_id(0)
    h = _fuse_h(acc_ref, den_ref, b2_ref)
    seg = batch_ref[...].reshape(RBLK, 1)
    onehot = (seg == lax.iota(jnp.int32, G).reshape(1, G)).astype(jnp.float32)
    ps = lax.dot_general(onehot, h, (((0,), (0,)), ((), ())),
                         preferred_element_type=jnp.float32)
    pc = lax.dot_general(onehot, jnp.ones((RBLK, F), jnp.float32),
                         (((0,), (0,)), ((), ())),
                         preferred_element_type=jnp.float32)

    @pl.when(i == 0)
    def _():
        psum_ref[...] = ps
        pcnt_ref[...] = pc

    @pl.when(i > 0)
    def _():
        psum_ref[...] += ps
        pcnt_ref[...] += pc

    @pl.when(i == pl.num_programs(0) - 1)
    def _():
        pooled = psum_ref[...] / jnp.maximum(pcnt_ref[...], 1.0)
        o_ref[...] = jnp.dot(pooled, wfc_ref[...],
                             preferred_element_type=jnp.float32) + bfc_ref[...]


def _final(acc, den3, b2, batch3, wfc, bfc):
    return pl.pallas_call(
        _final_body,
        grid=(NPAD // RBLK,),
        in_specs=[pl.BlockSpec((NC, RBLK, F), lambda i: (0, i, 0)),
                  pl.BlockSpec((NC, RBLK // F, F), lambda i: (0, i, 0)),
                  pl.BlockSpec((1, F), lambda i: (0, 0)),
                  pl.BlockSpec((1, 1, RBLK), lambda i: (i, 0, 0)),
                  pl.BlockSpec((F, F), lambda i: (0, 0)),
                  pl.BlockSpec((1, F), lambda i: (0, 0))],
        out_specs=pl.BlockSpec((G, F), lambda i: (0, 0)),
        out_shape=jax.ShapeDtypeStruct((G, F), jnp.float32),
        scratch_shapes=[pltpu.VMEM((G, F), jnp.float32),
                        pltpu.VMEM((G, F), jnp.float32)],
    )(acc, den3, b2.reshape(1, F), batch3, wfc, bfc.reshape(1, F))


def _edge_body(xl_hbm, xr_hbm, ef_hbm, src_hbm, dst_hbm, att_hbm,
               acc_hbm, den_hbm,
               src0_v, dst0_v, xj0_v, xi0_v, ef0_v,
               src1_v, dst1_v, xj1_v, xi1_v, ef1_v,
               dsc0_v, dsc1_v, w0_v, w1_v,
               t_v, att_v, zb_v, zd_v,
               acc_s, den_s, si0, si1, sg0, sg1, ss0, ss1):
    cid = lax.axis_index("c")
    sid = lax.axis_index("s")
    wid = cid * NS + sid
    sets = ((src0_v, dst0_v, xj0_v, xi0_v, ef0_v, dsc0_v, w0_v, si0, sg0, ss0),
            (src1_v, dst1_v, xj1_v, xi1_v, ef1_v, dsc1_v, w1_v, si1, sg1, ss1))

    def zr_body(r, carry):
        for k2 in range(F // L):
            zb_v[r, pl.ds(k2 * L, L)] = jnp.zeros((L,), jnp.float32)
        return carry

    lax.fori_loop(0, ZR, zr_body, 0)

    def zd_body(r, carry):
        zd_v[pl.ds(r * L, L)] = jnp.zeros((L,), jnp.float32)
        return carry

    lax.fori_loop(0, RPT // L, zd_body, 0)
    pltpu.sync_copy(zd_v, den_s.at[pl.ds(sid * RPT, RPT)])

    def zc_body(i, carry):
        pltpu.sync_copy(zb_v, acc_s.at[pl.ds(sid * RPT + i * ZR, ZR)])
        return carry

    lax.fori_loop(0, RPT // ZR, zc_body, 0)
    pltpu.sync_copy(att_hbm, att_v)
    for r in range(B, TPAD):
        t_v[r, :] = jnp.zeros((L,), jnp.float32)
    plsc.subcore_barrier()

    ebase = wid * EWP
    fiota = lax.iota(jnp.int32, L)
    att8 = tuple(att_v[pl.ds(k * L, L)] for k in range(F // L))

    def _issue_idx(c, s):
        src_v, dst_v = s[0], s[1]
        si = s[7]
        e0 = ebase + c * B
        pltpu.async_copy(src_hbm.at[pl.ds(e0, B)], src_v, si)
        pltpu.async_copy(dst_hbm.at[pl.ds(e0, B)], dst_v, si)

    def _wait_idx(c, s):
        src_v, dst_v = s[0], s[1]
        si = s[7]
        e0 = ebase + c * B
        pltpu.make_async_copy(src_hbm.at[pl.ds(e0, B)], src_v, si).wait()
        pltpu.make_async_copy(dst_hbm.at[pl.ds(e0, B)], dst_v, si).wait()

    def _issue_gather(c, s):
        src_v, dst_v, xj_v, xi_v, ef_v = s[0], s[1], s[2], s[3], s[4]
        sg = s[8]
        e0 = ebase + c * B
        pltpu.async_copy(xl_hbm.at[src_v], xj_v, sg)
        pltpu.async_copy(xr_hbm.at[dst_v], xi_v, sg)
        pltpu.async_copy(ef_hbm.at[pl.ds(e0, B)], ef_v, sg)

    def _wait_gather(c, s):
        src_v, dst_v, xj_v, xi_v, ef_v = s[0], s[1], s[2], s[3], s[4]
        sg = s[8]
        e0 = ebase + c * B
        pltpu.make_async_copy(xl_hbm.at[src_v], xj_v, sg).wait()
        pltpu.make_async_copy(xr_hbm.at[dst_v], xi_v, sg).wait()
        pltpu.make_async_copy(ef_hbm.at[pl.ds(e0, B)], ef_v, sg).wait()

    def _wait_scatter(s):
        xj_v, dsc_v, w_v, ss = s[2], s[5], s[6], s[9]
        pltpu.make_async_copy(xj_v, acc_s.at[dsc_v], ss).wait()
        pltpu.make_async_copy(w_v.at[pl.ds(0, B)], den_s.at[dsc_v], ss).wait()

    def _process(c, this, nxt, att8c):
        src_v, dst_v, xj_v, xi_v, ef_v, dsc_v, w_v = this[:7]
        ss = this[9]
        _wait_gather(c, this)

        @pl.when(c + 1 < NCH)
        def _():
            @pl.when(c >= 1)
            def _():
                _wait_scatter(nxt)
            _wait_idx(c + 1, nxt)
            _issue_gather(c + 1, nxt)

        @plsc.parallel_loop(0, B, step=1, unroll=4, carry=att8c)
        def e_loop(e, att8i):
            t = jnp.zeros((L,), jnp.float32)
            for k in range(F // L):
                s = (xi_v[e, pl.ds(k * L, L)] + xj_v[e, pl.ds(k * L, L)]
                     + ef_v[e, pl.ds(k * L, L)])
                m = jnp.where(s >= 0.0, s, 0.2 * s)
                t = t + m * att8i[k]
            t_v[e, :] = t
            return att8i

        att8c = e_loop

        for g in range(TPAD // L):
            rows = g * L + fiota
            lsum = jnp.zeros((L,), jnp.float32)
            for c16 in range(L):
                cols = jnp.full((L,), c16, jnp.int32)
                lsum = lsum + plsc.load_gather(t_v, [rows, cols])
            w_v[pl.ds(g * L, L)] = jnp.exp(lsum)

        @plsc.parallel_loop(0, B, step=1, unroll=4)
        def s_loop(e):
            w = w_v[pl.ds(e, L)][0]
            for k in range(F // L):
                xj_v[e, pl.ds(k * L, L)] = xj_v[e, pl.ds(k * L, L)] * w
        dsc_v[pl.ds(0, L)] = dst_v[pl.ds(0, L)]
        dsc_v[pl.ds(L, L)] = dst_v[pl.ds(L, L)]
        dsc_v[pl.ds(B - L, L)] = dst_v[pl.ds(B - L, L)]
        ss = this[9]
        pltpu.async_copy(xj_v, acc_s.at[dsc_v], ss, add=True)
        pltpu.async_copy(w_v.at[pl.ds(0, B)], den_s.at[dsc_v], ss, add=True)

        @pl.when(c + 2 < NCH)
        def _():
            _issue_idx(c + 2, this)

        return att8c

    _issue_idx(0, sets[0])
    _wait_idx(0, sets[0])
    _issue_gather(0, sets[0])
    _issue_idx(1, sets[1])

    def pair_body(cc, att8c):
        att8c = _process(2 * cc, sets[0], sets[1], att8c)
        att8c = _process(2 * cc + 1, sets[1], sets[0], att8c)
        return att8c

    lax.fori_loop(0, NCH // 2, pair_body, att8)
    _wait_scatter(sets[0])
    _wait_scatter(sets[1])
    plsc.subcore_barrier()

    def pub_body(i, carry):
        r0 = sid * RPT + i * ZR
        pltpu.sync_copy(acc_s.at[pl.ds(r0, ZR)], zb_v)
        pltpu.sync_copy(zb_v, acc_hbm.at[cid, pl.ds(r0, ZR)])
        return carry

    lax.fori_loop(0, RPT // ZR, pub_body, 0)
    pltpu.sync_copy(den_s.at[pl.ds(sid * RPT, RPT)], zd_v)
    pltpu.sync_copy(zd_v, den_hbm.at[cid, pl.ds(sid * RPT, RPT)])


def _edge_pass(xl, xr, ef, src, dst, att):
    mesh = plsc.VectorSubcoreMesh(core_axis_name="c", subcore_axis_name="s",
                                  num_cores=NC, num_subcores=NS)
    buf_set = [
        pltpu.VMEM((B,), jnp.int32),
        pltpu.VMEM((B,), jnp.int32),
        pltpu.VMEM((B, F), jnp.float32),
        pltpu.VMEM((B, F), jnp.float32),
        pltpu.VMEM((B, F), jnp.float32),
    ]
    kern = pl.kernel(
        _edge_body,
        out_type=(jax.ShapeDtypeStruct((NC, NPAD, F), jnp.float32),
                  jax.ShapeDtypeStruct((NC, NPAD), jnp.float32)),
        mesh=mesh,
        scratch_types=buf_set + buf_set + [
            pltpu.VMEM((B,), jnp.int32),
            pltpu.VMEM((B,), jnp.int32),
            pltpu.VMEM((TPAD + L,), jnp.float32),
            pltpu.VMEM((TPAD + L,), jnp.float32),
            pltpu.VMEM((TPAD, L), jnp.float32),
            pltpu.VMEM((F,), jnp.float32),
            pltpu.VMEM((ZR, F), jnp.float32),
            pltpu.VMEM((RPT,), jnp.float32),
            pltpu.VMEM_SHARED((NPAD, F), jnp.float32),
            pltpu.VMEM_SHARED((NPAD,), jnp.float32),
            pltpu.SemaphoreType.DMA,
            pltpu.SemaphoreType.DMA,
            pltpu.SemaphoreType.DMA,
            pltpu.SemaphoreType.DMA,
            pltpu.SemaphoreType.DMA,
            pltpu.SemaphoreType.DMA,
        ],
        compiler_params=pltpu.CompilerParams(use_tc_tiling_on_sc=False,
                                             needs_layout_passes=False),
    )
    return kern(xl, xr, ef, src, dst, att)


def kernel(x, edge_index, edge_attr, batch,
           Wl1, bl1, Wr1, br1, We1, att1, bias1,
           Wl2, bl2, Wr2, br2, We2, att2, bias2,
           Wfc, bfc):
    src = edge_index[0].astype(jnp.int32)
    dst = edge_index[1].astype(jnp.int32)
    ea_pad = edge_attr
    if EP > E:
        src = jnp.concatenate([src, jnp.zeros((EP - E,), jnp.int32)])
        dst = jnp.concatenate([dst, jnp.full((EP - E,), NPAD - 1, jnp.int32)])
        ea_pad = jnp.concatenate([edge_attr,
                                  jnp.zeros((EP - E, DE), jnp.float32)])
    batch_pad = jnp.concatenate(
        [batch.astype(jnp.int32), jnp.full((NPAD - N,), G, jnp.int32)])
    batch3 = batch_pad.reshape(NPAD // RBLK, 1, RBLK)

    ef1, xl1, xr1 = _prep(ea_pad, We1, x, Wl1, bl1, Wr1, br1)
    if EP > E:
        zrows = jnp.zeros((NPAD - N, F), jnp.float32)
        xl1 = jnp.concatenate([xl1, zrows])
        xr1 = jnp.concatenate([xr1, zrows])
    ef2 = _ef_one(ea_pad, We2)

    acc1, den1 = _edge_pass(xl1, xr1, ef1, src, dst, att1.reshape(F))
    xl2, xr2 = _mid(acc1, den1.reshape(NC, NPAD // F, F),
                    bias1, Wl2, bl2, Wr2, br2)
    acc2, den2 = _edge_pass(xl2, xr2, ef2, src, dst, att2.reshape(F))
    return _final(acc2, den2.reshape(NC, NPAD // F, F),
                  bias2, batch3, Wfc, bfc)
